# Initial kernel scaffold; baseline (speedup 1.0000x reference)
import jax
import jax.numpy as jnp
from jax.experimental import pallas as pl


def kernel(s_cat, s_cont, k_cat, k_cont, o_cat, o_cont, target, s_cat_tables, k_cat_tables, o_cat_tables, s_cont_vec, s_cont_bias, k_cont_vec, k_cont_bias, o_cont_vec, o_cont_bias, tgt_vec, tgt_bias):
    raise NotImplementedError("write your pallas kernel here")


# speedup vs baseline: 1.3480x; 1.3480x over previous
import functools

import jax
import jax.numpy as jnp
from jax import lax
from jax.experimental import pallas as pl
from jax.experimental.pallas import tpu as pltpu
from jax.experimental.pallas import tpu_sc as plsc

B, T, H = 1024, 50, 128
BT = B * T
KV = 1000
OV = 1000
SV = 100000
NC, NS = 2, 16
NW = NC * NS
ROWS_W = BT // NW
CHUNK = 80
NCH = ROWS_W // CHUNK
SROWS = B // NW
NG = 6


def _sc_gather(cat_f, scat_f, k_tab, o_tab, s_tab):
    mesh = plsc.VectorSubcoreMesh(core_axis_name="c", subcore_axis_name="s")

    @functools.partial(
        pl.kernel,
        out_type=(
            jax.ShapeDtypeStruct((BT, 12 * H), jnp.float32),
            jax.ShapeDtypeStruct((BT, 8 * H), jnp.float32),
            jax.ShapeDtypeStruct((B, 6 * H), jnp.float32),
        ),
        mesh=mesh,
        scratch_types=[
            pltpu.VMEM((NG, CHUNK), jnp.int32),
            pltpu.VMEM((NG, CHUNK, H), jnp.float32),
            pltpu.VMEM((SROWS,), jnp.int32),
            pltpu.VMEM((SROWS, H), jnp.float32),
            pltpu.SemaphoreType.DMA,
        ],
    )
    def body(cat_hbm, scat_hbm, ktab_hbm, otab_hbm, stab_hbm,
             kout_hbm, oout_hbm, sout_hbm,
             idx_v, rows_v, sidx_v, srows_v, sem):
        wid = lax.axis_index("s") * NC + lax.axis_index("c")

        sbase = wid * SROWS
        for i in range(2):
            pltpu.sync_copy(scat_hbm.at[pl.ds(i * B + sbase, SROWS)], sidx_v)
            if i:
                for v in range(SROWS // 16):
                    sl = pl.ds(v * 16, 16)
                    sidx_v[sl] = sidx_v[sl] + i * SV
            pltpu.async_copy(stab_hbm.at[sidx_v], srows_v, sem).wait()
            pltpu.sync_copy(
                srows_v, sout_hbm.at[pl.ds(sbase, SROWS), pl.ds(i * H, H)])

        def chunk_body(c, carry):
            base = wid * ROWS_W + c * CHUNK
            for g in range(NG):
                pltpu.sync_copy(
                    cat_hbm.at[pl.ds(g * BT + base, CHUNK)], idx_v.at[g])
            for g in range(NG):
                off = g * KV if g < 4 else (g - 4) * OV
                if off:
                    for v in range(CHUNK // 16):
                        sl = pl.ds(v * 16, 16)
                        idx_v[g, sl] = idx_v[g, sl] + off
            descs = []
            for g in range(NG):
                tab = ktab_hbm if g < 4 else otab_hbm
                descs.append(
                    pltpu.async_copy(tab.at[idx_v.at[g]], rows_v.at[g], sem))
            for d in descs:
                d.wait()
            descs = []
            for g in range(NG):
                if g < 4:
                    dst = kout_hbm.at[pl.ds(base, CHUNK), pl.ds(g * H, H)]
                else:
                    dst = oout_hbm.at[pl.ds(base, CHUNK), pl.ds((g - 4) * H, H)]
                descs.append(pltpu.async_copy(rows_v.at[g], dst, sem))
            for d in descs:
                d.wait()
            return carry

        lax.fori_loop(0, NCH, chunk_body, 0)

    return body(cat_f, scat_f, k_tab, o_tab, s_tab)


def _cont_body(nv_step, nsteps, c_ref, vec_ref, bias_ref, alias_ref, out_ref):
    j = pl.program_id(1)
    c = c_ref[...]
    vec = vec_ref[...]
    bias = bias_ref[...]
    for jj in range(nsteps):

        @pl.when(j == jj)
        def _():
            for v in range(nv_step):
                cv = jj * nv_step + v
                out_ref[:, pl.ds(v * H, H)] = (
                    c[:, cv:cv + 1] * vec[cv][None, :] + bias[cv][None, :])


def _cont_fill(cont2, vec, bias, cat_buf, ncat, nv_step, rch):
    n_rows, ncont = cont2.shape
    nsteps = ncont // nv_step
    ncols = (ncat + ncont) * H
    return pl.pallas_call(
        functools.partial(_cont_body, nv_step, nsteps),
        grid=(n_rows // rch, nsteps),
        in_specs=[
            pl.BlockSpec((rch, ncont), lambda i, j: (i, 0)),
            pl.BlockSpec((ncont, H), lambda i, j: (0, 0)),
            pl.BlockSpec((ncont, H), lambda i, j: (0, 0)),
            pl.BlockSpec((8, 128), lambda i, j: (0, 0)),
        ],
        out_specs=pl.BlockSpec(
            (rch, nv_step * H), lambda i, j: (i, j + ncat // nv_step)),
        out_shape=jax.ShapeDtypeStruct((n_rows, ncols), jnp.float32),
        input_output_aliases={3: 0},
    )(cont2, vec, bias, cat_buf)


def _tgt_body(c_ref, vec_ref, bias_ref, out_ref):
    out_ref[...] = (c_ref[...] * vec_ref[...][0][None, :]
                    + bias_ref[...][0][None, :])


def _tgt_fill(cont2, vec, bias, rch):
    n_rows = cont2.shape[0]
    return pl.pallas_call(
        _tgt_body,
        grid=(n_rows // rch,),
        in_specs=[
            pl.BlockSpec((rch, 1), lambda i: (i, 0)),
            pl.BlockSpec((1, H), lambda i: (0, 0)),
            pl.BlockSpec((1, H), lambda i: (0, 0)),
        ],
        out_specs=pl.BlockSpec((rch, H), lambda i: (i, 0)),
        out_shape=jax.ShapeDtypeStruct((n_rows, H), jnp.float32),
    )(cont2, vec, bias)


def kernel(s_cat, s_cont, k_cat, k_cont, o_cat, o_cont, target,
           s_cat_tables, k_cat_tables, o_cat_tables,
           s_cont_vec, s_cont_bias, k_cont_vec, k_cont_bias,
           o_cont_vec, o_cont_bias, tgt_vec, tgt_bias):
    kcat_t = k_cat.reshape(BT, 4).T.reshape(-1)
    ocat_t = o_cat.reshape(BT, 2).T.reshape(-1)
    cat_f = jnp.concatenate([kcat_t, ocat_t])
    scat_f = s_cat[:, 0, :].T.reshape(-1)
    k_tab = k_cat_tables.reshape(4 * KV, H)
    o_tab = o_cat_tables.reshape(2 * OV, H)
    s_tab = s_cat_tables.reshape(2 * SV, H)

    k_buf, o_buf, s_buf = _sc_gather(cat_f, scat_f, k_tab, o_tab, s_tab)

    k_full = _cont_fill(k_cont.reshape(BT, 8), k_cont_vec, k_cont_bias,
                        k_buf, 4, 4, 512)
    o_full = _cont_fill(o_cont.reshape(BT, 6), o_cont_vec, o_cont_bias,
                        o_buf, 2, 2, 512)
    s_full = _cont_fill(s_cont[:, 0, :], s_cont_vec, s_cont_bias,
                        s_buf, 2, 2, 512)
    t_full = _tgt_fill(target.reshape(BT, 1), tgt_vec, tgt_bias, 512)

    return (s_full.reshape(B, 6, H),
            k_full.reshape(B, T, 12, H),
            o_full.reshape(B, T, 8, H),
            t_full.reshape(B, T, 1, H))


# speedup vs baseline: 1.6186x; 1.2008x over previous
import functools

import jax
import jax.numpy as jnp
from jax import lax
from jax.experimental import pallas as pl
from jax.experimental.pallas import tpu as pltpu
from jax.experimental.pallas import tpu_sc as plsc

B, T, H = 1024, 50, 128
BT = B * T
KV = 1000
OV = 1000
SV = 100000
NC, NS = 2, 16
NW = NC * NS
ROWS_W = BT // NW
CHUNK = 80
NCH = ROWS_W // CHUNK
SROWS = B // NW
NG = 6


def _sc_gather(cat_f, scat_f, k_tab, o_tab, s_tab):
    mesh = plsc.VectorSubcoreMesh(core_axis_name="c", subcore_axis_name="s")

    @functools.partial(
        pl.kernel,
        out_type=(
            jax.ShapeDtypeStruct((BT, 4 * H), jnp.float32),
            jax.ShapeDtypeStruct((BT, 8 * H), jnp.float32),
            jax.ShapeDtypeStruct((B, 2 * H), jnp.float32),
        ),
        mesh=mesh,
        scratch_types=[
            pltpu.VMEM((NG, CHUNK), jnp.int32),
            pltpu.VMEM((NG, CHUNK, H), jnp.float32),
            pltpu.VMEM((SROWS,), jnp.int32),
            pltpu.VMEM((SROWS, H), jnp.float32),
            pltpu.SemaphoreType.DMA,
        ],
    )
    def body(cat_hbm, scat_hbm, ktab_hbm, otab_hbm, stab_hbm,
             kout_hbm, oout_hbm, sout_hbm,
             idx_v, rows_v, sidx_v, srows_v, sem):
        wid = lax.axis_index("s") * NC + lax.axis_index("c")

        sbase = wid * SROWS
        for i in range(2):
            pltpu.sync_copy(scat_hbm.at[pl.ds(i * B + sbase, SROWS)], sidx_v)
            if i:
                for v in range(SROWS // 16):
                    sl = pl.ds(v * 16, 16)
                    sidx_v[sl] = sidx_v[sl] + i * SV
            pltpu.async_copy(stab_hbm.at[sidx_v], srows_v, sem).wait()
            pltpu.sync_copy(
                srows_v, sout_hbm.at[pl.ds(sbase, SROWS), pl.ds(i * H, H)])

        def chunk_body(c, carry):
            base = wid * ROWS_W + c * CHUNK
            for g in range(NG):
                pltpu.sync_copy(
                    cat_hbm.at[pl.ds(g * BT + base, CHUNK)], idx_v.at[g])
            for g in range(NG):
                off = g * KV if g < 4 else (g - 4) * OV
                if off:
                    for v in range(CHUNK // 16):
                        sl = pl.ds(v * 16, 16)
                        idx_v[g, sl] = idx_v[g, sl] + off
            descs = []
            for g in range(NG):
                tab = ktab_hbm if g < 4 else otab_hbm
                descs.append(
                    pltpu.async_copy(tab.at[idx_v.at[g]], rows_v.at[g], sem))
            for d in descs:
                d.wait()
            descs = []
            for g in range(NG):
                if g < 4:
                    dst = kout_hbm.at[pl.ds(base, CHUNK), pl.ds(g * H, H)]
                else:
                    dst = oout_hbm.at[pl.ds(base, CHUNK), pl.ds((g - 4) * H, H)]
                descs.append(pltpu.async_copy(rows_v.at[g], dst, sem))
            for d in descs:
                d.wait()
            return carry

        lax.fori_loop(0, NCH, chunk_body, 0)

    return body(cat_f, scat_f, k_tab, o_tab, s_tab)


def _cont_body(nv_step, nsteps, c_ref, vec_ref, bias_ref, alias_ref, out_ref):
    j = pl.program_id(1)
    c = c_ref[...]
    vec = vec_ref[...]
    bias = bias_ref[...]
    for jj in range(nsteps):

        @pl.when(j == jj)
        def _():
            for v in range(nv_step):
                cv = jj * nv_step + v
                out_ref[:, pl.ds(v * H, H)] = (
                    c[:, cv:cv + 1] * vec[cv][None, :] + bias[cv][None, :])


def _cont_fill(cont2, vec, bias, cat_buf, ncat, nv_step, rch):
    n_rows, ncont = cont2.shape
    nsteps = ncont // nv_step
    ncols = (ncat + ncont) * H
    return pl.pallas_call(
        functools.partial(_cont_body, nv_step, nsteps),
        grid=(n_rows // rch, nsteps),
        in_specs=[
            pl.BlockSpec((rch, ncont), lambda i, j: (i, 0)),
            pl.BlockSpec((ncont, H), lambda i, j: (0, 0)),
            pl.BlockSpec((ncont, H), lambda i, j: (0, 0)),
            pl.BlockSpec((8, 128), lambda i, j: (0, 0)),
        ],
        out_specs=pl.BlockSpec(
            (rch, nv_step * H), lambda i, j: (i, j + ncat // nv_step)),
        out_shape=jax.ShapeDtypeStruct((n_rows, ncols), jnp.float32),
        input_output_aliases={3: 0},
    )(cont2, vec, bias, cat_buf)


GBK = 8
GBT = 32
SB = 256


def _known_body(cat_ref, c_ref, vec_ref, bias_ref, out_ref):
    c = c_ref[...]
    vec = vec_ref[...]
    bias = bias_ref[...]
    for v in range(4):
        out_ref[:, :, v, :] = cat_ref[:, pl.ds(v * H, H)].reshape(GBK, T, H)
    for cv in range(8):
        out_ref[:, :, 4 + cv, :] = (
            c[:, cv:cv + 1] * vec[cv][None, :] + bias[cv][None, :]
        ).reshape(GBK, T, H)


def _known_fill(cat_buf, cont2, vec, bias):
    return pl.pallas_call(
        _known_body,
        grid=(B // GBK,),
        in_specs=[
            pl.BlockSpec((GBK * T, 4 * H), lambda i: (i, 0)),
            pl.BlockSpec((GBK * T, 8), lambda i: (i, 0)),
            pl.BlockSpec((8, H), lambda i: (0, 0)),
            pl.BlockSpec((8, H), lambda i: (0, 0)),
        ],
        out_specs=pl.BlockSpec((GBK, T, 12, H), lambda i: (i, 0, 0, 0)),
        out_shape=jax.ShapeDtypeStruct((B, T, 12, H), jnp.float32),
    )(cat_buf, cont2, vec, bias)


def _static_body(cat_ref, c_ref, vec_ref, bias_ref, out_ref):
    c = c_ref[...]
    vec = vec_ref[...]
    bias = bias_ref[...]
    for v in range(2):
        out_ref[:, v, :] = cat_ref[:, pl.ds(v * H, H)]
    for cv in range(4):
        out_ref[:, 2 + cv, :] = (
            c[:, cv:cv + 1] * vec[cv][None, :] + bias[cv][None, :])


def _static_fill(cat_buf, cont2, vec, bias):
    return pl.pallas_call(
        _static_body,
        grid=(B // SB,),
        in_specs=[
            pl.BlockSpec((SB, 2 * H), lambda i: (i, 0)),
            pl.BlockSpec((SB, 4), lambda i: (i, 0)),
            pl.BlockSpec((4, H), lambda i: (0, 0)),
            pl.BlockSpec((4, H), lambda i: (0, 0)),
        ],
        out_specs=pl.BlockSpec((SB, 6, H), lambda i: (i, 0, 0)),
        out_shape=jax.ShapeDtypeStruct((B, 6, H), jnp.float32),
    )(cat_buf, cont2, vec, bias)


def _tgt_body(c_ref, vec_ref, bias_ref, out_ref):
    out_ref[:, :, 0, :] = (
        c_ref[...] * vec_ref[...][0][None, :] + bias_ref[...][0][None, :]
    ).reshape(GBT, T, H)


def _tgt_fill(cont2, vec, bias):
    return pl.pallas_call(
        _tgt_body,
        grid=(B // GBT,),
        in_specs=[
            pl.BlockSpec((GBT * T, 1), lambda i: (i, 0)),
            pl.BlockSpec((1, H), lambda i: (0, 0)),
            pl.BlockSpec((1, H), lambda i: (0, 0)),
        ],
        out_specs=pl.BlockSpec((GBT, T, 1, H), lambda i: (i, 0, 0, 0)),
        out_shape=jax.ShapeDtypeStruct((B, T, 1, H), jnp.float32),
    )(cont2, vec, bias)


def kernel(s_cat, s_cont, k_cat, k_cont, o_cat, o_cont, target,
           s_cat_tables, k_cat_tables, o_cat_tables,
           s_cont_vec, s_cont_bias, k_cont_vec, k_cont_bias,
           o_cont_vec, o_cont_bias, tgt_vec, tgt_bias):
    kcat_t = k_cat.reshape(BT, 4).T.reshape(-1)
    ocat_t = o_cat.reshape(BT, 2).T.reshape(-1)
    cat_f = jnp.concatenate([kcat_t, ocat_t])
    scat_f = s_cat[:, 0, :].T.reshape(-1)
    k_tab = k_cat_tables.reshape(4 * KV, H)
    o_tab = o_cat_tables.reshape(2 * OV, H)
    s_tab = s_cat_tables.reshape(2 * SV, H)

    kcat_buf, o_buf, scat_buf = _sc_gather(cat_f, scat_f, k_tab, o_tab, s_tab)

    k_full = _known_fill(kcat_buf, k_cont.reshape(BT, 8),
                         k_cont_vec, k_cont_bias)
    o_full = _cont_fill(o_cont.reshape(BT, 6), o_cont_vec, o_cont_bias,
                        o_buf, 2, 2, 512)
    s_full = _static_fill(scat_buf, s_cont[:, 0, :],
                          s_cont_vec, s_cont_bias)
    t_full = _tgt_fill(target.reshape(BT, 1), tgt_vec, tgt_bias)

    return (s_full,
            k_full,
            o_full.reshape(B, T, 8, H),
            t_full)
